# baseline (speedup 1.0000x reference)
import functools

import jax
import jax.numpy as jnp
from jax import lax
from jax.experimental import pallas as pl
from jax.experimental.pallas import tpu as pltpu
from jax.experimental.pallas import tpu_sc as plsc

_SC_PARAMS = dict(
    compiler_params=pltpu.CompilerParams(
        use_tc_tiling_on_sc=True, needs_layout_passes=False),
)


@functools.lru_cache(maxsize=None)
def _make_transpose(V, D, NC, NS):
    NW = NC * NS
    RW = 128
    PACK = RW // D
    n_full = V // RW
    tail = V - n_full * RW
    mesh = plsc.VectorSubcoreMesh(core_axis_name="c", subcore_axis_name="s")

    def transpose_block(src, dst, width):
        for q in range(width // PACK):
            for a in range(PACK):
                col = jnp.full((16,), PACK * q + a, jnp.int32)
                for h in range(D // 16):
                    vec = plsc.load_gather(
                        src, [lax.iota(jnp.int32, 16) + 16 * h, col])
                    dst[q, pl.ds(D * a + 16 * h, 16)] = vec

    @functools.partial(
        pl.kernel,
        mesh=mesh,
        out_type=jax.ShapeDtypeStruct((V * D // RW, RW), jnp.float32),
        scratch_types=[
            pltpu.VMEM((2, D, RW), jnp.float32),
            pltpu.VMEM((2, RW // PACK, RW), jnp.float32),
            pltpu.SemaphoreType.DMA((2,)),
            pltpu.SemaphoreType.DMA((2,)),
        ],
        **_SC_PARAMS,
    )
    def transpose_kernel(tab_hbm, tail_hbm, out_hbm, sbuf, obuf, isem, osem):
        wid = lax.axis_index("s") * NC + lax.axis_index("c")
        c0 = n_full * wid // NW
        nw = n_full * (wid + 1) // NW - c0

        def prime(m, p):
            pltpu.async_copy(
                tab_hbm.at[:, pl.ds((c0 + m) * RW, RW)], sbuf.at[p],
                isem.at[p])

        def handle(m, p, first):
            pltpu.make_async_copy(
                tab_hbm.at[:, pl.ds(0, RW)], sbuf.at[p], isem.at[p]).wait()
            if not first:
                pltpu.make_async_copy(
                    obuf.at[p], out_hbm.at[pl.ds(0, RW // PACK)], osem.at[p]
                ).wait()
            transpose_block(sbuf.at[p], obuf.at[p], RW)
            pltpu.async_copy(
                obuf.at[p],
                out_hbm.at[pl.ds((c0 + m) * (RW // PACK), RW // PACK)],
                osem.at[p])

        prime(0, 0)
        prime(1, 1)
        handle(0, 0, True)
        prime(2, 0)
        handle(1, 1, True)
        prime(3, 1)

        def body(g2, carry):
            for u in range(2):
                m = 2 * g2 + u
                handle(m, u, False)

                @pl.when(m + 2 < nw)
                def _():
                    prime(m + 2, u)

            return carry

        lax.fori_loop(1, 122, body, 0)

        @pl.when(nw > 244)
        def _():
            handle(244, 0, False)

        for p in range(2):
            pltpu.make_async_copy(
                obuf.at[p], out_hbm.at[pl.ds(0, RW // PACK)], osem.at[p]
            ).wait()

        if tail:
            @pl.when(wid == NW - 1)
            def _():
                pltpu.sync_copy(tail_hbm, sbuf.at[0, pl.ds(0, tail // PACK)])
                pltpu.sync_copy(
                    sbuf.at[0, pl.ds(0, tail // PACK)],
                    out_hbm.at[pl.ds(n_full * (RW // PACK), tail // PACK)])

    return transpose_kernel


@functools.lru_cache(maxsize=None)
def _make_gather(BATCH, FIELDS, D, NC, NS):
    NW = NC * NS
    BB = 128
    n_blocks = BATCH * FIELDS // BB
    bpw = n_blocks // NW
    RW = 128
    PACK = RW // D
    mesh = plsc.VectorSubcoreMesh(core_axis_name="c", subcore_axis_name="s")

    @functools.partial(
        pl.kernel,
        mesh=mesh,
        out_type=jax.ShapeDtypeStruct((FIELDS, D, BATCH), jnp.float32),
        scratch_types=[
            pltpu.VMEM((bpw * BB,), jnp.int32),
            pltpu.VMEM((2, BB), jnp.int32),
            pltpu.VMEM((2, BB), jnp.int32),
            pltpu.VMEM((2, BB, RW), jnp.float32),
            pltpu.VMEM((2, D, BB), jnp.float32),
            pltpu.SemaphoreType.DMA((2,)),
            pltpu.SemaphoreType.DMA((2,)),
        ],
        **_SC_PARAMS,
    )
    def gather_kernel(idx_hbm, table_hbm, out_hbm, idxall, rowv, offv,
                      gbuf, obuf, gsem, ssem):
        wid = lax.axis_index("s") * NC + lax.axis_index("c")
        blk0 = wid * bpw
        pltpu.sync_copy(idx_hbm.at[pl.ds(blk0 * BB, bpw * BB)], idxall)
        iotas = [lax.iota(jnp.int32, 16) + 16 * i for i in range(BB // 16)]

        def prime(g, p):
            for i in range(BB // 16):
                v = idxall[pl.ds(g * BB + 16 * i, 16)]
                rowv[p, pl.ds(16 * i, 16)] = v >> 2
                offv[p, pl.ds(16 * i, 16)] = (v & 3) << 5
            pltpu.async_copy(table_hbm.at[rowv.at[p]], gbuf.at[p], gsem.at[p])

        def wait_gather(p):
            pltpu.make_async_copy(
                table_hbm.at[rowv.at[p]], gbuf.at[p], gsem.at[p]).wait()

        def assemble_store(g, p, first):
            if not first:
                pltpu.make_async_copy(
                    obuf.at[p], out_hbm.at[0, :, pl.ds(0, BB)], ssem.at[p]
                ).wait()
            bases = [offv[p, pl.ds(16 * i, 16)] for i in range(BB // 16)]
            for d in range(D):
                for i in range(BB // 16):
                    vec = plsc.load_gather(
                        gbuf.at[p], [iotas[i], bases[i] + d])
                    obuf[p, d, pl.ds(16 * i, 16)] = vec
            blk = blk0 + g
            j = blk // (BATCH // BB)
            cb = blk % (BATCH // BB)
            pltpu.async_copy(
                obuf.at[p], out_hbm.at[j, :, pl.ds(cb * BB, BB)], ssem.at[p])

        prime(0, 0)
        prime(1, 1)
        wait_gather(0)
        assemble_store(0, 0, first=True)
        prime(2, 0)
        wait_gather(1)
        assemble_store(1, 1, first=True)
        prime(3, 1)

        def body(g2, carry):
            for u in range(2):
                g = 2 * g2 + u
                wait_gather(u)
                assemble_store(g, u, first=False)
                prime(g + 2, u)
            return carry

        lax.fori_loop(1, bpw // 2 - 1, body, 0)
        for u in range(2):
            g = bpw - 2 + u
            wait_gather(u)
            assemble_store(g, u, first=False)
        for p in range(2):
            pltpu.make_async_copy(
                obuf.at[p], out_hbm.at[0, :, pl.ds(0, BB)], ssem.at[p]).wait()

    return gather_kernel


def kernel(input, embedding_matrix):
    BATCH, FIELDS = input.shape
    V, D = embedding_matrix.shape
    info = plsc.get_sparse_core_info()
    NC, NS = info.num_cores, info.num_subcores
    idx_fb = input.T.reshape(BATCH * FIELDS).astype(jnp.int32)
    n_full = V // (128 // D) // 128 * 128
    tail_rm = embedding_matrix[n_full * (128 // D):].reshape(-1, 128)
    table_rm = _make_transpose(V, D, NC, NS)(embedding_matrix.T, tail_rm)
    out = _make_gather(BATCH, FIELDS, D, NC, NS)(idx_fb, table_rm)
    return jnp.transpose(out, (2, 0, 1))


# speedup vs baseline: 4.4657x; 4.4657x over previous
import functools

import jax
import jax.numpy as jnp
from jax import lax
from jax.experimental import pallas as pl
from jax.experimental.pallas import tpu as pltpu
from jax.experimental.pallas import tpu_sc as plsc

_SC_PARAMS = dict(
    compiler_params=pltpu.CompilerParams(
        use_tc_tiling_on_sc=True, needs_layout_passes=False),
)


@functools.lru_cache(maxsize=None)
def _make_transpose(V, D, NC, NS):
    NW = NC * NS
    RW = 128
    PACK = RW // D
    n_full = V // RW
    tail = V - n_full * RW
    mesh = plsc.VectorSubcoreMesh(core_axis_name="c", subcore_axis_name="s")

    def transpose_block(src, dst, width):
        iotas = [lax.iota(jnp.int32, 16) + 16 * h for h in range(D // 16)]

        @functools.partial(plsc.parallel_loop, 0, width // PACK, unroll=4)
        def _(q):
            for a in range(PACK):
                col = jnp.full((16,), 0, jnp.int32) + (PACK * q + a)
                for h in range(D // 16):
                    vec = plsc.load_gather(src, [iotas[h], col])
                    dst[q, pl.ds(D * a + 16 * h, 16)] = vec

    @functools.partial(
        pl.kernel,
        mesh=mesh,
        out_type=jax.ShapeDtypeStruct((V * D // RW, RW), jnp.float32),
        scratch_types=[
            pltpu.VMEM((2, D, RW), jnp.float32),
            pltpu.VMEM((2, RW // PACK, RW), jnp.float32),
            pltpu.SemaphoreType.DMA((2,)),
            pltpu.SemaphoreType.DMA((2,)),
        ],
        **_SC_PARAMS,
    )
    def transpose_kernel(tab_hbm, tail_hbm, out_hbm, sbuf, obuf, isem, osem):
        wid = lax.axis_index("s") * NC + lax.axis_index("c")
        c0 = n_full * wid // NW
        nw = n_full * (wid + 1) // NW - c0

        def prime(m, p):
            pltpu.async_copy(
                tab_hbm.at[:, pl.ds((c0 + m) * RW, RW)], sbuf.at[p],
                isem.at[p])

        def handle(m, p, first):
            pltpu.make_async_copy(
                tab_hbm.at[:, pl.ds(0, RW)], sbuf.at[p], isem.at[p]).wait()
            if not first:
                pltpu.make_async_copy(
                    obuf.at[p], out_hbm.at[pl.ds(0, RW // PACK)], osem.at[p]
                ).wait()
            transpose_block(sbuf.at[p], obuf.at[p], RW)
            pltpu.async_copy(
                obuf.at[p],
                out_hbm.at[pl.ds((c0 + m) * (RW // PACK), RW // PACK)],
                osem.at[p])

        prime(0, 0)
        prime(1, 1)
        handle(0, 0, True)
        prime(2, 0)
        handle(1, 1, True)
        prime(3, 1)

        def body(g2, carry):
            for u in range(2):
                m = 2 * g2 + u
                handle(m, u, False)

                @pl.when(m + 2 < nw)
                def _():
                    prime(m + 2, u)

            return carry

        lax.fori_loop(1, 122, body, 0)

        @pl.when(nw > 244)
        def _():
            handle(244, 0, False)

        for p in range(2):
            pltpu.make_async_copy(
                obuf.at[p], out_hbm.at[pl.ds(0, RW // PACK)], osem.at[p]
            ).wait()

        if tail:
            @pl.when(wid == NW - 1)
            def _():
                pltpu.sync_copy(tail_hbm, sbuf.at[0, pl.ds(0, tail // PACK)])
                pltpu.sync_copy(
                    sbuf.at[0, pl.ds(0, tail // PACK)],
                    out_hbm.at[pl.ds(n_full * (RW // PACK), tail // PACK)])

    return transpose_kernel


@functools.lru_cache(maxsize=None)
def _make_gather(BATCH, FIELDS, D, NC, NS):
    NW = NC * NS
    BB = 128
    n_blocks = BATCH * FIELDS // BB
    bpw = n_blocks // NW
    RW = 128
    PACK = RW // D
    mesh = plsc.VectorSubcoreMesh(core_axis_name="c", subcore_axis_name="s")

    @functools.partial(
        pl.kernel,
        mesh=mesh,
        out_type=jax.ShapeDtypeStruct((FIELDS, D, BATCH), jnp.float32),
        scratch_types=[
            pltpu.VMEM((bpw * BB,), jnp.int32),
            pltpu.VMEM((2, BB), jnp.int32),
            pltpu.VMEM((2, BB), jnp.int32),
            pltpu.VMEM((2, BB, RW), jnp.float32),
            pltpu.VMEM((2, D, BB), jnp.float32),
            pltpu.SemaphoreType.DMA((2,)),
            pltpu.SemaphoreType.DMA((2,)),
        ],
        **_SC_PARAMS,
    )
    def gather_kernel(idx_hbm, table_hbm, out_hbm, idxall, rowv, offv,
                      gbuf, obuf, gsem, ssem):
        wid = lax.axis_index("s") * NC + lax.axis_index("c")
        blk0 = wid * bpw
        pltpu.sync_copy(idx_hbm.at[pl.ds(blk0 * BB, bpw * BB)], idxall)
        iotas = [lax.iota(jnp.int32, 16) + 16 * i for i in range(BB // 16)]

        def prime(g, p):
            for i in range(BB // 16):
                v = idxall[pl.ds(g * BB + 16 * i, 16)]
                rowv[p, pl.ds(16 * i, 16)] = v >> 2
                offv[p, pl.ds(16 * i, 16)] = (v & 3) << 5
            pltpu.async_copy(table_hbm.at[rowv.at[p]], gbuf.at[p], gsem.at[p])

        def wait_gather(p):
            pltpu.make_async_copy(
                table_hbm.at[rowv.at[p]], gbuf.at[p], gsem.at[p]).wait()

        def assemble_store(g, p, first):
            if not first:
                pltpu.make_async_copy(
                    obuf.at[p], out_hbm.at[0, :, pl.ds(0, BB)], ssem.at[p]
                ).wait()
            bases = [offv[p, pl.ds(16 * i, 16)] for i in range(BB // 16)]

            @functools.partial(plsc.parallel_loop, 0, D, unroll=4)
            def _(d):
                for i in range(BB // 16):
                    vec = plsc.load_gather(
                        gbuf.at[p], [iotas[i], bases[i] + d])
                    obuf[p, d, pl.ds(16 * i, 16)] = vec
            blk = blk0 + g
            j = blk // (BATCH // BB)
            cb = blk % (BATCH // BB)
            pltpu.async_copy(
                obuf.at[p], out_hbm.at[j, :, pl.ds(cb * BB, BB)], ssem.at[p])

        prime(0, 0)
        prime(1, 1)
        wait_gather(0)
        assemble_store(0, 0, first=True)
        prime(2, 0)
        wait_gather(1)
        assemble_store(1, 1, first=True)
        prime(3, 1)

        def body(g2, carry):
            for u in range(2):
                g = 2 * g2 + u
                wait_gather(u)
                assemble_store(g, u, first=False)
                prime(g + 2, u)
            return carry

        lax.fori_loop(1, bpw // 2 - 1, body, 0)
        for u in range(2):
            g = bpw - 2 + u
            wait_gather(u)
            assemble_store(g, u, first=False)
        for p in range(2):
            pltpu.make_async_copy(
                obuf.at[p], out_hbm.at[0, :, pl.ds(0, BB)], ssem.at[p]).wait()

    return gather_kernel


def kernel(input, embedding_matrix):
    BATCH, FIELDS = input.shape
    V, D = embedding_matrix.shape
    info = plsc.get_sparse_core_info()
    NC, NS = info.num_cores, info.num_subcores
    idx_fb = input.T.reshape(BATCH * FIELDS).astype(jnp.int32)
    n_full = V // (128 // D) // 128 * 128
    tail_rm = embedding_matrix[n_full * (128 // D):].reshape(-1, 128)
    table_rm = _make_transpose(V, D, NC, NS)(embedding_matrix.T, tail_rm)
    out = _make_gather(BATCH, FIELDS, D, NC, NS)(idx_fb, table_rm)
    return jnp.transpose(out, (2, 0, 1))
